# baseline (speedup 1.0000x reference)
import functools

import jax
import jax.numpy as jnp
from jax.experimental import pallas as pl
from jax.experimental.pallas import tpu as pltpu


def _nbytes(shape, dtype):
    n = 1
    for d in shape:
        n *= int(d)
    return n * jnp.dtype(dtype).itemsize


def _vlim(est_bytes):
    est = int(1.3 * est_bytes) + (6 << 20)
    return min(max(est, 16 * 1024 * 1024), 56 * 1024 * 1024)


def _stem_kernel(a_ref, w_ref, s_ref, t_ref, o_ref, pp_ref, *, Nh, Ho, Wo):
    y = jnp.dot(a_ref[...], w_ref[...], preferred_element_type=jnp.float32)
    y = jnp.maximum(y * s_ref[...] + t_ref[...], 0.0).astype(jnp.bfloat16)
    C = y.shape[-1]
    Hp, Wp = Ho // 2, Wo // 2
    pp_ref[...] = jnp.zeros(pp_ref.shape, pp_ref.dtype)
    pp_ref[:, 1:Ho + 1, 1:Wo + 1, :] = y.reshape(Nh, Ho, Wo, C)
    rm = jnp.maximum(jnp.maximum(pp_ref[:, 0:Ho, :, :], pp_ref[:, 1:Ho + 1, :, :]),
                     pp_ref[:, 2:Ho + 2, :, :])
    re = rm.reshape(Nh, Hp, 2, Wo + 2, C)[:, :, 0]
    cm = jnp.maximum(jnp.maximum(re[:, :, 0:Wo, :], re[:, :, 1:Wo + 1, :]),
                     re[:, :, 2:Wo + 2, :])
    ce = cm.reshape(Nh, Hp, Wp, 2, C)[:, :, :, 0]
    o_ref[...] = ce.reshape(Nh * Hp * Wp, C)


def _stem_call(a, w, s, t, N, Ho, Wo):
    M, K = a.shape
    C = w.shape[1]
    Nh = N // 2
    Mh = M // 2
    Hp, Wp = Ho // 2, Wo // 2
    Mo = N * Hp * Wp
    est = (_nbytes((Mh, K), jnp.bfloat16) * 2 + _nbytes((K, C), jnp.bfloat16)
           + _nbytes((Nh, Ho + 2, Wo + 2, C), jnp.bfloat16)
           + _nbytes((Mo // 2, C), jnp.bfloat16) * 2)
    return pl.pallas_call(
        functools.partial(_stem_kernel, Nh=Nh, Ho=Ho, Wo=Wo),
        out_shape=jax.ShapeDtypeStruct((Mo, C), jnp.bfloat16),
        grid_spec=pltpu.PrefetchScalarGridSpec(
            num_scalar_prefetch=0,
            grid=(2,),
            in_specs=[
                pl.BlockSpec((Mh, K), lambda c: (c, 0)),
                pl.BlockSpec((K, C), lambda c: (0, 0)),
                pl.BlockSpec((1, C), lambda c: (0, 0)),
                pl.BlockSpec((1, C), lambda c: (0, 0)),
            ],
            out_specs=pl.BlockSpec((Mo // 2, C), lambda c: (c, 0)),
            scratch_shapes=[pltpu.VMEM((Nh, Ho + 2, Wo + 2, C), jnp.bfloat16)],
        ),
        compiler_params=pltpu.CompilerParams(
            dimension_semantics=("parallel",),
            vmem_limit_bytes=_vlim(est),
        ),
    )(a, w, s, t)


def _block_compute(xv, wr, pad_ref, col_ref, *, Nh, H, W, P, Cin, stride,
                   has_down):
    if has_down:
        w1, s1, t1, w2, s2, t2, w3, s3, t3, wd, sd, td = wr
    else:
        w1, s1, t1, w2, s2, t2, w3, s3, t3 = wr
    Ho, Wo = H // stride, W // stride

    y1 = jnp.dot(xv, w1[0], preferred_element_type=jnp.float32)
    y1 = jnp.maximum(y1 * s1[0] + t1[0], 0.0).astype(jnp.bfloat16)

    pad_ref[:, 1:H + 1, 1:W + 1, :] = y1.reshape(Nh, H, W, P)
    taps = []
    for di in range(3):
        for dj in range(3):
            tap = pad_ref[:, di:di + H, dj:dj + W, :]
            if stride == 2:
                tap = tap.reshape(Nh, Ho, 2, Wo, 2, P)[:, :, 0, :, 0, :]
            taps.append(tap.reshape(Nh * Ho * Wo, P))
    if col_ref is not None:
        for ti, tp in enumerate(taps):
            col_ref[:, ti * P:(ti + 1) * P] = tp
        y2 = jnp.dot(col_ref[...], w2[0][0:9 * P, :],
                     preferred_element_type=jnp.float32)
    else:
        y2 = None
        for ti, tp in enumerate(taps):
            d = jnp.dot(tp, w2[0][ti * P:(ti + 1) * P, :],
                        preferred_element_type=jnp.float32)
            y2 = d if y2 is None else y2 + d
    y2 = jnp.maximum(y2 * s2[0] + t2[0], 0.0).astype(jnp.bfloat16)

    y3 = jnp.dot(y2, w3[0], preferred_element_type=jnp.float32)
    y3 = y3 * s3[0] + t3[0]

    if has_down:
        if stride == 2:
            xd = xv.reshape(Nh, Ho, 2, Wo, 2, Cin)[:, :, 0, :, 0, :]
            xd = xd.reshape(Nh * Ho * Wo, Cin)
        else:
            xd = xv
        r = jnp.dot(xd, wd[0], preferred_element_type=jnp.float32)
        r = r * sd[0] + td[0]
    else:
        r = xv.astype(jnp.float32)
    return jnp.maximum(y3 + r, 0.0).astype(jnp.bfloat16)


def _stage_kernel(*refs, cfgs, Nh, use_col, share_pad):
    nb = len(cfgs)
    x_ref = refs[0]
    pos = 1
    wrefs = []
    for k, (H, W, P, Cin, stride, has_down) in enumerate(cfgs):
        n = 12 if has_down else 9
        wrefs.append(refs[pos:pos + n])
        pos += n
    o_ref = refs[pos]
    act_ref = refs[pos + 1]
    pad0_ref = refs[pos + 2]
    padt_ref = pad0_ref if share_pad else refs[pos + 3]
    col_ref = refs[pos + (3 if share_pad else 4)] if use_col else None

    i = pl.program_id(1)

    @pl.when(i == 0)
    def _zero_pads():
        pad0_ref[...] = jnp.zeros(pad0_ref.shape, pad0_ref.dtype)
        if not share_pad:
            padt_ref[...] = jnp.zeros(padt_ref.shape, padt_ref.dtype)

    for k in range(nb):
        H, W, P, Cin, stride, has_down = cfgs[k]

        def _arm(k=k, H=H, W=W, P=P, Cin=Cin, stride=stride,
                 has_down=has_down):
            xv = x_ref[...] if k == 0 else act_ref[...]
            out = _block_compute(
                xv, wrefs[k], pad0_ref if k == 0 else padt_ref, col_ref,
                Nh=Nh, H=H, W=W, P=P, Cin=Cin, stride=stride,
                has_down=has_down)
            if k == nb - 1:
                o_ref[...] = out
            else:
                act_ref[...] = out

        pl.when(i == k)(_arm)


def _stage_call(h, blocks, N, H, W):
    M, Cin0 = h.shape
    nb = len(blocks)
    Nh = N // 2
    Mh = M // 2
    P = blocks[0][0][0].shape[1]
    Cout = blocks[0][2][0].shape[1]
    stride0 = blocks[0][4]
    Ho, Wo = H // stride0, W // stride0
    Mho = N * Ho * Wo // 2
    use_col = (P % 128 == 0)
    share_pad = (stride0 == 1)

    args = [h]
    in_specs = [pl.BlockSpec((Mh, Cin0), lambda c, i: (c, 0))]
    cfgs = []
    est = _nbytes((Mh, Cin0), jnp.bfloat16)
    Hk, Wk, Ck = H, W, Cin0
    for (c1, c2, c3, down, stride) in blocks:
        has_down = down is not None
        cfgs.append((Hk, Wk, P, Ck, stride, has_down))
        group = [c1[0], c1[1], c1[2], c2[0], c2[1], c2[2], c3[0], c3[1], c3[2]]
        if has_down:
            group += [down[0], down[1], down[2]]
        for arr in group:
            args.append(arr)
            sh = arr.shape
            in_specs.append(
                pl.BlockSpec((1,) + sh,
                             lambda c, i, _n=len(sh) + 1: (0,) * _n))
            est += _nbytes(sh, arr.dtype)
        Hk, Wk, Ck = Hk // stride, Wk // stride, Cout

    scratch = [
        pltpu.VMEM((Mho, Cout), jnp.bfloat16),
        pltpu.VMEM((Nh, H + 2, W + 2, P), jnp.bfloat16),
    ]
    est += 2 * _nbytes((Mho, Cout), jnp.bfloat16) * 2
    est += _nbytes((Nh, H + 2, W + 2, P), jnp.bfloat16)
    if not share_pad:
        scratch.append(pltpu.VMEM((Nh, Ho + 2, Wo + 2, P), jnp.bfloat16))
        est += _nbytes((Nh, Ho + 2, Wo + 2, P), jnp.bfloat16)
    if use_col:
        scratch.append(pltpu.VMEM((Mho, 9 * P), jnp.bfloat16))
        est += _nbytes((Mho, 9 * P), jnp.bfloat16)

    args = [args[0]] + [a.reshape((1,) + a.shape) for a in args[1:]]

    out = pl.pallas_call(
        functools.partial(_stage_kernel, cfgs=cfgs, Nh=Nh, use_col=use_col,
                          share_pad=share_pad),
        out_shape=jax.ShapeDtypeStruct((N * Ho * Wo, Cout), jnp.bfloat16),
        grid_spec=pltpu.PrefetchScalarGridSpec(
            num_scalar_prefetch=0,
            grid=(2, nb),
            in_specs=in_specs,
            out_specs=pl.BlockSpec((Mho, Cout), lambda c, i: (c, 0)),
            scratch_shapes=scratch,
        ),
        compiler_params=pltpu.CompilerParams(
            dimension_semantics=("parallel", "arbitrary"),
            vmem_limit_bytes=_vlim(est),
        ),
    )(*args)
    return out


def _fc_kernel(x_ref, w_ref, s_ref, t_ref, o_ref, *, N, HW):
    feat = x_ref[...].astype(jnp.float32).reshape(N, HW, x_ref.shape[-1])
    feat = feat.mean(axis=1)
    y = jnp.dot(feat.astype(jnp.bfloat16), w_ref[...],
                preferred_element_type=jnp.float32)
    y = y * s_ref[...] + t_ref[...]
    o_ref[...] = jax.nn.sigmoid(y)


def _fc_call(h, w, s, t, N, HW):
    M, C = h.shape
    Cout = w.shape[1]
    est = (_nbytes((M, C), jnp.bfloat16) + _nbytes(w.shape, jnp.bfloat16)
           + _nbytes((N, Cout), jnp.float32))
    return pl.pallas_call(
        functools.partial(_fc_kernel, N=N, HW=HW),
        grid_spec=pltpu.PrefetchScalarGridSpec(
            num_scalar_prefetch=0,
            grid=(1,),
            in_specs=[
                pl.BlockSpec((M, C), lambda i: (0, 0)),
                pl.BlockSpec((C, Cout), lambda i: (0, 0)),
                pl.BlockSpec((1, Cout), lambda i: (0, 0)),
                pl.BlockSpec((1, Cout), lambda i: (0, 0)),
            ],
            out_specs=pl.BlockSpec((N, Cout), lambda i: (0, 0)),
        ),
        out_shape=jax.ShapeDtypeStruct((N, Cout), jnp.float32),
        compiler_params=pltpu.CompilerParams(
            dimension_semantics=("arbitrary",),
            vmem_limit_bytes=_vlim(est),
        ),
    )(h, w, s, t)


def _stem_patches(x, Kp):
    xh = jnp.transpose(x, (0, 2, 3, 1)).astype(jnp.bfloat16)
    N, H, W, C = xh.shape
    xp = jnp.pad(xh, ((0, 0), (3, 3), (3, 3), (0, 0)))
    Ho, Wo = H // 2, W // 2
    taps = [xp[:, i:i + 2 * Ho:2, j:j + 2 * Wo:2, :]
            for i in range(7) for j in range(7)]
    a = jnp.stack(taps, axis=3).reshape(N * Ho * Wo, 49 * C)
    if a.shape[1] < Kp:
        a = jnp.pad(a, ((0, 0), (0, Kp - a.shape[1])))
    return a, Ho, Wo


def kernel(*args):
    a = list(args)
    x = a[0]
    conv1 = a[1:4]
    idx = 4
    nblocks = [3, 4, 6, 3]
    layers = []
    for L in range(4):
        blocks = []
        for b in range(nblocks[L]):
            c1 = a[idx:idx + 3]
            c2 = a[idx + 3:idx + 6]
            c3 = a[idx + 6:idx + 9]
            idx += 9
            down = None
            if b == 0:
                down = a[idx:idx + 3]
                idx += 3
            stride = 2 if (L > 0 and b == 0) else 1
            blocks.append((c1, c2, c3, down, stride))
        layers.append(blocks)
    fc_w, fc_scale, fc_shift = a[idx:idx + 3]

    N = x.shape[0]
    patches, Ho, Wo = _stem_patches(x, conv1[0].shape[0])
    h = _stem_call(patches, conv1[0], conv1[1], conv1[2], N, Ho, Wo)
    H = W = Ho // 2
    for blocks in layers:
        h = _stage_call(h, blocks, N, H, W)
        stride0 = blocks[0][4]
        H, W = H // stride0, W // stride0
    out = _fc_call(h, fc_w, fc_scale, fc_shift, N, H * W)
    return out[:, :1]


# speedup vs baseline: 1.7806x; 1.7806x over previous
import functools

import jax
import jax.numpy as jnp
from jax.experimental import pallas as pl
from jax.experimental.pallas import tpu as pltpu


def _nbytes(shape, dtype):
    n = 1
    for d in shape:
        n *= int(d)
    return n * jnp.dtype(dtype).itemsize


def _vlim(est_bytes):
    est = int(1.3 * est_bytes) + (6 << 20)
    return min(max(est, 16 * 1024 * 1024), 56 * 1024 * 1024)


def _stem_kernel(a_ref, w_ref, s_ref, t_ref, o_ref, pp_ref, *, Nh, Ho, Wo):
    y = None
    for di in range(4):
        for dj in range(4):
            tap = a_ref[:, di:di + Ho, dj:dj + Wo, :].reshape(Nh * Ho * Wo, 12)
            d = jnp.dot(tap, w_ref[di * 4 + dj],
                        preferred_element_type=jnp.float32)
            y = d if y is None else y + d
    y = jnp.maximum(y * s_ref[...] + t_ref[...], 0.0).astype(jnp.bfloat16)
    C = y.shape[-1]
    Hp, Wp = Ho // 2, Wo // 2
    pp_ref[...] = jnp.zeros(pp_ref.shape, pp_ref.dtype)
    pp_ref[:, 1:Ho + 1, 1:Wo + 1, :] = y.reshape(Nh, Ho, Wo, C)
    rm = jnp.maximum(jnp.maximum(pp_ref[:, 0:Ho, :, :], pp_ref[:, 1:Ho + 1, :, :]),
                     pp_ref[:, 2:Ho + 2, :, :])
    re = rm.reshape(Nh, Hp, 2, Wo + 2, C)[:, :, 0]
    cm = jnp.maximum(jnp.maximum(re[:, :, 0:Wo, :], re[:, :, 1:Wo + 1, :]),
                     re[:, :, 2:Wo + 2, :])
    ce = cm.reshape(Nh, Hp, Wp, 2, C)[:, :, :, 0]
    o_ref[...] = ce.reshape(Nh * Hp * Wp, C)


def _stem_call(a, w16, s, t, N, Ho, Wo):
    C = w16.shape[2]
    Nh = N // 2
    Hp, Wp = Ho // 2, Wo // 2
    Mo = N * Hp * Wp
    est = (_nbytes(a.shape, jnp.bfloat16) + _nbytes(w16.shape, jnp.bfloat16)
           + _nbytes((Nh, Ho + 2, Wo + 2, C), jnp.bfloat16)
           + _nbytes((Mo // 2, C), jnp.bfloat16) * 2)
    return pl.pallas_call(
        functools.partial(_stem_kernel, Nh=Nh, Ho=Ho, Wo=Wo),
        out_shape=jax.ShapeDtypeStruct((Mo, C), jnp.bfloat16),
        grid_spec=pltpu.PrefetchScalarGridSpec(
            num_scalar_prefetch=0,
            grid=(2,),
            in_specs=[
                pl.BlockSpec((Nh, Ho + 3, Wo + 3, 12), lambda c: (c, 0, 0, 0)),
                pl.BlockSpec(w16.shape, lambda c: (0, 0, 0)),
                pl.BlockSpec((1, C), lambda c: (0, 0)),
                pl.BlockSpec((1, C), lambda c: (0, 0)),
            ],
            out_specs=pl.BlockSpec((Mo // 2, C), lambda c: (c, 0)),
            scratch_shapes=[pltpu.VMEM((Nh, Ho + 2, Wo + 2, C), jnp.bfloat16)],
        ),
        compiler_params=pltpu.CompilerParams(
            dimension_semantics=("parallel",),
            vmem_limit_bytes=_vlim(est),
        ),
    )(a, w16, s, t)


def _block_compute(xv, wr, pad_ref, col_ref, *, Nh, H, W, P, Cin, stride,
                   has_down):
    if has_down:
        w1, s1, t1, w2, s2, t2, w3, s3, t3, wd, sd, td = wr
    else:
        w1, s1, t1, w2, s2, t2, w3, s3, t3 = wr
    Ho, Wo = H // stride, W // stride

    y1 = jnp.dot(xv, w1[0], preferred_element_type=jnp.float32)
    y1 = jnp.maximum(y1 * s1[0] + t1[0], 0.0).astype(jnp.bfloat16)

    pad_ref[:, 1:H + 1, 1:W + 1, :] = y1.reshape(Nh, H, W, P)
    taps = []
    for di in range(3):
        for dj in range(3):
            tap = pad_ref[:, di:di + H, dj:dj + W, :]
            if stride == 2:
                tap = tap.reshape(Nh, Ho, 2, Wo, 2, P)[:, :, 0, :, 0, :]
            taps.append(tap.reshape(Nh * Ho * Wo, P))
    if col_ref is not None:
        for ti, tp in enumerate(taps):
            col_ref[:, ti * P:(ti + 1) * P] = tp
        y2 = jnp.dot(col_ref[...], w2[0][0:9 * P, :],
                     preferred_element_type=jnp.float32)
    else:
        y2 = None
        for ti, tp in enumerate(taps):
            d = jnp.dot(tp, w2[0][ti * P:(ti + 1) * P, :],
                        preferred_element_type=jnp.float32)
            y2 = d if y2 is None else y2 + d
    y2 = jnp.maximum(y2 * s2[0] + t2[0], 0.0).astype(jnp.bfloat16)

    y3 = jnp.dot(y2, w3[0], preferred_element_type=jnp.float32)
    y3 = y3 * s3[0] + t3[0]

    if has_down:
        if stride == 2:
            xd = xv.reshape(Nh, Ho, 2, Wo, 2, Cin)[:, :, 0, :, 0, :]
            xd = xd.reshape(Nh * Ho * Wo, Cin)
        else:
            xd = xv
        r = jnp.dot(xd, wd[0], preferred_element_type=jnp.float32)
        r = r * sd[0] + td[0]
    else:
        r = xv.astype(jnp.float32)
    return jnp.maximum(y3 + r, 0.0).astype(jnp.bfloat16)


def _stage_kernel(*refs, cfgs, Nh, use_col, share_pad):
    nb = len(cfgs)
    x_ref = refs[0]
    pos = 1
    wrefs = []
    for k, (H, W, P, Cin, stride, has_down) in enumerate(cfgs):
        n = 12 if has_down else 9
        wrefs.append(refs[pos:pos + n])
        pos += n
    o_ref = refs[pos]
    act_ref = refs[pos + 1]
    pad0_ref = refs[pos + 2]
    padt_ref = pad0_ref if share_pad else refs[pos + 3]
    col_ref = refs[pos + (3 if share_pad else 4)] if use_col else None

    i = pl.program_id(1)

    @pl.when(i == 0)
    def _zero_pads():
        pad0_ref[...] = jnp.zeros(pad0_ref.shape, pad0_ref.dtype)
        if not share_pad:
            padt_ref[...] = jnp.zeros(padt_ref.shape, padt_ref.dtype)

    for k in range(nb):
        H, W, P, Cin, stride, has_down = cfgs[k]

        def _arm(k=k, H=H, W=W, P=P, Cin=Cin, stride=stride,
                 has_down=has_down):
            xv = x_ref[...] if k == 0 else act_ref[...]
            out = _block_compute(
                xv, wrefs[k], pad0_ref if k == 0 else padt_ref, col_ref,
                Nh=Nh, H=H, W=W, P=P, Cin=Cin, stride=stride,
                has_down=has_down)
            if k == nb - 1:
                o_ref[...] = out
            else:
                act_ref[...] = out

        pl.when(i == k)(_arm)


def _stage_call(h, blocks, N, H, W):
    M, Cin0 = h.shape
    nb = len(blocks)
    Nh = N // 2
    Mh = M // 2
    P = blocks[0][0][0].shape[1]
    Cout = blocks[0][2][0].shape[1]
    stride0 = blocks[0][4]
    Ho, Wo = H // stride0, W // stride0
    Mho = N * Ho * Wo // 2
    use_col = (P % 128 == 0)
    share_pad = (stride0 == 1)

    args = [h]
    in_specs = [pl.BlockSpec((Mh, Cin0), lambda c, i: (c, 0))]
    cfgs = []
    est = _nbytes((Mh, Cin0), jnp.bfloat16)
    Hk, Wk, Ck = H, W, Cin0
    for (c1, c2, c3, down, stride) in blocks:
        has_down = down is not None
        cfgs.append((Hk, Wk, P, Ck, stride, has_down))
        group = [c1[0], c1[1], c1[2], c2[0], c2[1], c2[2], c3[0], c3[1], c3[2]]
        if has_down:
            group += [down[0], down[1], down[2]]
        for arr in group:
            args.append(arr)
            sh = arr.shape
            in_specs.append(
                pl.BlockSpec((1,) + sh,
                             lambda c, i, _n=len(sh) + 1: (0,) * _n))
            est += _nbytes(sh, arr.dtype)
        Hk, Wk, Ck = Hk // stride, Wk // stride, Cout

    scratch = [
        pltpu.VMEM((Mho, Cout), jnp.bfloat16),
        pltpu.VMEM((Nh, H + 2, W + 2, P), jnp.bfloat16),
    ]
    est += 2 * _nbytes((Mho, Cout), jnp.bfloat16) * 2
    est += _nbytes((Nh, H + 2, W + 2, P), jnp.bfloat16)
    if not share_pad:
        scratch.append(pltpu.VMEM((Nh, Ho + 2, Wo + 2, P), jnp.bfloat16))
        est += _nbytes((Nh, Ho + 2, Wo + 2, P), jnp.bfloat16)
    if use_col:
        scratch.append(pltpu.VMEM((Mho, 9 * P), jnp.bfloat16))
        est += _nbytes((Mho, 9 * P), jnp.bfloat16)

    args = [args[0]] + [a.reshape((1,) + a.shape) for a in args[1:]]

    out = pl.pallas_call(
        functools.partial(_stage_kernel, cfgs=cfgs, Nh=Nh, use_col=use_col,
                          share_pad=share_pad),
        out_shape=jax.ShapeDtypeStruct((N * Ho * Wo, Cout), jnp.bfloat16),
        grid_spec=pltpu.PrefetchScalarGridSpec(
            num_scalar_prefetch=0,
            grid=(2, nb),
            in_specs=in_specs,
            out_specs=pl.BlockSpec((Mho, Cout), lambda c, i: (c, 0)),
            scratch_shapes=scratch,
        ),
        compiler_params=pltpu.CompilerParams(
            dimension_semantics=("parallel", "arbitrary"),
            vmem_limit_bytes=_vlim(est),
        ),
    )(*args)
    return out


def _fc_kernel(x_ref, w_ref, s_ref, t_ref, o_ref, *, N, HW):
    feat = x_ref[...].astype(jnp.float32).reshape(N, HW, x_ref.shape[-1])
    feat = feat.mean(axis=1)
    y = jnp.dot(feat.astype(jnp.bfloat16), w_ref[...],
                preferred_element_type=jnp.float32)
    y = y * s_ref[...] + t_ref[...]
    o_ref[...] = jax.nn.sigmoid(y)


def _fc_call(h, w, s, t, N, HW):
    M, C = h.shape
    Cout = w.shape[1]
    est = (_nbytes((M, C), jnp.bfloat16) + _nbytes(w.shape, jnp.bfloat16)
           + _nbytes((N, Cout), jnp.float32))
    return pl.pallas_call(
        functools.partial(_fc_kernel, N=N, HW=HW),
        grid_spec=pltpu.PrefetchScalarGridSpec(
            num_scalar_prefetch=0,
            grid=(1,),
            in_specs=[
                pl.BlockSpec((M, C), lambda i: (0, 0)),
                pl.BlockSpec((C, Cout), lambda i: (0, 0)),
                pl.BlockSpec((1, Cout), lambda i: (0, 0)),
                pl.BlockSpec((1, Cout), lambda i: (0, 0)),
            ],
            out_specs=pl.BlockSpec((N, Cout), lambda i: (0, 0)),
        ),
        out_shape=jax.ShapeDtypeStruct((N, Cout), jnp.float32),
        compiler_params=pltpu.CompilerParams(
            dimension_semantics=("arbitrary",),
            vmem_limit_bytes=_vlim(est),
        ),
    )(h, w, s, t)


def _stem_prep(x, w):
    xh = jnp.transpose(x, (0, 2, 3, 1)).astype(jnp.bfloat16)
    N, H, W, _ = xh.shape
    xp = jnp.pad(xh, ((0, 0), (3, 3), (3, 3), (0, 0)))
    Hp = (H + 6) // 2
    xsd = xp.reshape(N, Hp, 2, Hp, 2, 3).transpose(0, 1, 3, 2, 4, 5)
    xsd = xsd.reshape(N, Hp, Hp, 12)
    perm = []
    for di in range(4):
        for dj in range(4):
            for p in range(2):
                for q in range(2):
                    for c in range(3):
                        i, j = 2 * di + p, 2 * dj + q
                        perm.append(21 * i + 3 * j + c
                                    if (i < 7 and j < 7) else w.shape[0] - 1)
    w16 = w[jnp.array(perm, jnp.int32)].reshape(16, 12, w.shape[1])
    return xsd, w16, H // 2, W // 2


def kernel(*args):
    a = list(args)
    x = a[0]
    conv1 = a[1:4]
    idx = 4
    nblocks = [3, 4, 6, 3]
    layers = []
    for L in range(4):
        blocks = []
        for b in range(nblocks[L]):
            c1 = a[idx:idx + 3]
            c2 = a[idx + 3:idx + 6]
            c3 = a[idx + 6:idx + 9]
            idx += 9
            down = None
            if b == 0:
                down = a[idx:idx + 3]
                idx += 3
            stride = 2 if (L > 0 and b == 0) else 1
            blocks.append((c1, c2, c3, down, stride))
        layers.append(blocks)
    fc_w, fc_scale, fc_shift = a[idx:idx + 3]

    N = x.shape[0]
    xsd, w16, Ho, Wo = _stem_prep(x, conv1[0])
    h = _stem_call(xsd, w16, conv1[1], conv1[2], N, Ho, Wo)
    H = W = Ho // 2
    for blocks in layers:
        h = _stage_call(h, blocks, N, H, W)
        stride0 = blocks[0][4]
        H, W = H // stride0, W // stride0
    out = _fc_call(h, fc_w, fc_scale, fc_shift, N, H * W)
    return out[:, :1]


# speedup vs baseline: 1.8269x; 1.0260x over previous
import functools

import jax
import jax.numpy as jnp
from jax.experimental import pallas as pl
from jax.experimental.pallas import tpu as pltpu


def _nbytes(shape, dtype):
    n = 1
    for d in shape:
        n *= int(d)
    return n * jnp.dtype(dtype).itemsize


def _vlim(est_bytes):
    est = int(1.3 * est_bytes) + (6 << 20)
    return min(max(est, 16 * 1024 * 1024), 56 * 1024 * 1024)


def _stem_kernel(a_ref, w_ref, s_ref, t_ref, o_ref, pp_ref, *, Nh, Ho, Wo):
    y = None
    for di in range(4):
        for dj in range(4):
            tap = a_ref[:, di:di + Ho, dj:dj + Wo, :].reshape(Nh * Ho * Wo, 12)
            d = jnp.dot(tap, w_ref[di * 4 + dj],
                        preferred_element_type=jnp.float32)
            y = d if y is None else y + d
    y = jnp.maximum(y * s_ref[...] + t_ref[...], 0.0).astype(jnp.bfloat16)
    C = y.shape[-1]
    Hp, Wp = Ho // 2, Wo // 2
    pp_ref[...] = jnp.zeros(pp_ref.shape, pp_ref.dtype)
    pp_ref[:, 1:Ho + 1, 1:Wo + 1, :] = y.reshape(Nh, Ho, Wo, C)
    rm = jnp.maximum(jnp.maximum(pp_ref[:, 0:Ho, :, :], pp_ref[:, 1:Ho + 1, :, :]),
                     pp_ref[:, 2:Ho + 2, :, :])
    re = rm.reshape(Nh, Hp, 2, Wo + 2, C)[:, :, 0]
    cm = jnp.maximum(jnp.maximum(re[:, :, 0:Wo, :], re[:, :, 1:Wo + 1, :]),
                     re[:, :, 2:Wo + 2, :])
    ce = cm.reshape(Nh, Hp, Wp, 2, C)[:, :, :, 0]
    o_ref[...] = ce.reshape(Nh * Hp * Wp, C)


def _stem_call(a, w16, s, t, N, Ho, Wo):
    C = w16.shape[2]
    Nh = N // 2
    Hp, Wp = Ho // 2, Wo // 2
    Mo = N * Hp * Wp
    est = (_nbytes(a.shape, jnp.bfloat16) + _nbytes(w16.shape, jnp.bfloat16)
           + _nbytes((Nh, Ho + 2, Wo + 2, C), jnp.bfloat16)
           + _nbytes((Mo // 2, C), jnp.bfloat16) * 2)
    return pl.pallas_call(
        functools.partial(_stem_kernel, Nh=Nh, Ho=Ho, Wo=Wo),
        out_shape=jax.ShapeDtypeStruct((Mo, C), jnp.bfloat16),
        grid_spec=pltpu.PrefetchScalarGridSpec(
            num_scalar_prefetch=0,
            grid=(2,),
            in_specs=[
                pl.BlockSpec((Nh, Ho + 3, Wo + 3, 12), lambda c: (c, 0, 0, 0)),
                pl.BlockSpec(w16.shape, lambda c: (0, 0, 0)),
                pl.BlockSpec((1, C), lambda c: (0, 0)),
                pl.BlockSpec((1, C), lambda c: (0, 0)),
            ],
            out_specs=pl.BlockSpec((Mo // 2, C), lambda c: (c, 0)),
            scratch_shapes=[pltpu.VMEM((Nh, Ho + 2, Wo + 2, C), jnp.bfloat16)],
        ),
        compiler_params=pltpu.CompilerParams(
            dimension_semantics=("parallel",),
            vmem_limit_bytes=_vlim(est),
        ),
    )(a, w16, s, t)


def _block_compute(xv, wr, pad_ref, col_ref, *, Nh, H, W, P, Cin, stride,
                   has_down):
    if has_down:
        w1, s1, t1, w2, s2, t2, w3, s3, t3, wd, sd, td = wr
    else:
        w1, s1, t1, w2, s2, t2, w3, s3, t3 = wr
    Ho, Wo = H // stride, W // stride

    y1 = jnp.dot(xv, w1[...], preferred_element_type=jnp.float32)
    y1 = jnp.maximum(y1 * s1[...] + t1[...], 0.0).astype(jnp.bfloat16)

    pad_ref[:, 1:H + 1, 1:W + 1, :] = y1.reshape(Nh, H, W, P)
    taps = []
    for di in range(3):
        for dj in range(3):
            tap = pad_ref[:, di:di + H, dj:dj + W, :]
            if stride == 2:
                tap = tap.reshape(Nh, Ho, 2, Wo, 2, P)[:, :, 0, :, 0, :]
            taps.append(tap.reshape(Nh * Ho * Wo, P))
    if col_ref is not None:
        for ti, tp in enumerate(taps):
            col_ref[:, ti * P:(ti + 1) * P] = tp
        y2 = jnp.dot(col_ref[...], w2[0:9 * P, :],
                     preferred_element_type=jnp.float32)
    else:
        y2 = None
        for ti, tp in enumerate(taps):
            d = jnp.dot(tp, w2[ti * P:(ti + 1) * P, :],
                        preferred_element_type=jnp.float32)
            y2 = d if y2 is None else y2 + d
    y2 = jnp.maximum(y2 * s2[...] + t2[...], 0.0).astype(jnp.bfloat16)

    y3 = jnp.dot(y2, w3[...], preferred_element_type=jnp.float32)
    y3 = y3 * s3[...] + t3[...]

    if has_down:
        if stride == 2:
            xd = xv.reshape(Nh, Ho, 2, Wo, 2, Cin)[:, :, 0, :, 0, :]
            xd = xd.reshape(Nh * Ho * Wo, Cin)
        else:
            xd = xv
        r = jnp.dot(xd, wd[...], preferred_element_type=jnp.float32)
        r = r * sd[...] + td[...]
    else:
        r = xv.astype(jnp.float32)
    return jnp.maximum(y3 + r, 0.0).astype(jnp.bfloat16)


def _stage_kernel(*refs, cfgs, Nh, use_col, share_pad):
    nb = len(cfgs)
    nbig = sum(4 if hd else 3 for (_, _, _, _, _, hd) in cfgs)
    x_ref = refs[0]
    pos = 1
    hbm_ws = refs[pos:pos + nbig]
    pos += nbig
    smalls = []
    for (_, _, _, _, _, hd) in cfgs:
        n = 8 if hd else 6
        smalls.append(refs[pos:pos + n])
        pos += n
    o_ref = refs[pos]
    pos += 1
    vmem_ws = refs[pos:pos + nbig]
    pos += nbig
    sem = refs[pos]
    act_ref = refs[pos + 1]
    pad0_ref = refs[pos + 2]
    padt_ref = pad0_ref if share_pad else refs[pos + 3]
    col_ref = refs[pos + (3 if share_pad else 4)] if use_col else None

    copies = []
    for k in range(nbig):
        cp = pltpu.make_async_copy(hbm_ws[k], vmem_ws[k], sem.at[k])
        cp.start()
        copies.append(cp)

    pad0_ref[...] = jnp.zeros(pad0_ref.shape, pad0_ref.dtype)
    if not share_pad:
        padt_ref[...] = jnp.zeros(padt_ref.shape, padt_ref.dtype)

    wi = 0
    for k in range(nb):
        H, W, P, Cin, stride, has_down = cfgs[k]
        nw = 4 if has_down else 3
        for j in range(nw):
            copies[wi + j].wait()
        if has_down:
            s1, t1, s2, t2, s3, t3, sd, td = smalls[k]
            wr = (vmem_ws[wi], s1, t1, vmem_ws[wi + 1], s2, t2,
                  vmem_ws[wi + 2], s3, t3, vmem_ws[wi + 3], sd, td)
        else:
            s1, t1, s2, t2, s3, t3 = smalls[k]
            wr = (vmem_ws[wi], s1, t1, vmem_ws[wi + 1], s2, t2,
                  vmem_ws[wi + 2], s3, t3)
        wi += nw
        xv = x_ref[...] if k == 0 else act_ref[...]
        out = _block_compute(xv, wr, pad0_ref if k == 0 else padt_ref,
                             col_ref, Nh=Nh, H=H, W=W, P=P, Cin=Cin,
                             stride=stride, has_down=has_down)
        if k == nb - 1:
            o_ref[...] = out
        else:
            act_ref[...] = out


def _stage_call(h, blocks, N, H, W):
    M, Cin0 = h.shape
    nb = len(blocks)
    Nh = N // 2
    Mh = M // 2
    P = blocks[0][0][0].shape[1]
    Cout = blocks[0][2][0].shape[1]
    stride0 = blocks[0][4]
    Ho, Wo = H // stride0, W // stride0
    Mho = N * Ho * Wo // 2
    use_col = (P % 128 == 0)
    share_pad = (stride0 == 1)

    big_args, small_args = [], []
    cfgs = []
    est = _nbytes((Mh, Cin0), jnp.bfloat16) * 2
    Hk, Wk, Ck = H, W, Cin0
    for (c1, c2, c3, down, stride) in blocks:
        has_down = down is not None
        cfgs.append((Hk, Wk, P, Ck, stride, has_down))
        bigs = [c1[0], c2[0], c3[0]] + ([down[0]] if has_down else [])
        sm = [c1[1], c1[2], c2[1], c2[2], c3[1], c3[2]]
        if has_down:
            sm += [down[1], down[2]]
        big_args += bigs
        small_args += sm
        for arr in bigs:
            est += 2 * _nbytes(arr.shape, arr.dtype)
        for arr in sm:
            est += _nbytes(arr.shape, arr.dtype)
        Hk, Wk, Ck = Hk // stride, Wk // stride, Cout

    in_specs = ([pl.BlockSpec((Mh, Cin0), lambda c: (c, 0))]
                + [pl.BlockSpec(memory_space=pl.ANY) for _ in big_args]
                + [pl.BlockSpec(arr.shape, lambda c: (0, 0))
                   for arr in small_args])

    scratch = [pltpu.VMEM(arr.shape, arr.dtype) for arr in big_args]
    scratch.append(pltpu.SemaphoreType.DMA((len(big_args),)))
    scratch += [
        pltpu.VMEM((Mho, Cout), jnp.bfloat16),
        pltpu.VMEM((Nh, H + 2, W + 2, P), jnp.bfloat16),
    ]
    est += 2 * _nbytes((Mho, Cout), jnp.bfloat16) * 2
    est += _nbytes((Nh, H + 2, W + 2, P), jnp.bfloat16)
    if not share_pad:
        scratch.append(pltpu.VMEM((Nh, Ho + 2, Wo + 2, P), jnp.bfloat16))
        est += _nbytes((Nh, Ho + 2, Wo + 2, P), jnp.bfloat16)
    if use_col:
        scratch.append(pltpu.VMEM((Mho, 9 * P), jnp.bfloat16))
        est += _nbytes((Mho, 9 * P), jnp.bfloat16)

    out = pl.pallas_call(
        functools.partial(_stage_kernel, cfgs=cfgs, Nh=Nh, use_col=use_col,
                          share_pad=share_pad),
        out_shape=jax.ShapeDtypeStruct((N * Ho * Wo, Cout), jnp.bfloat16),
        grid_spec=pltpu.PrefetchScalarGridSpec(
            num_scalar_prefetch=0,
            grid=(2,),
            in_specs=in_specs,
            out_specs=pl.BlockSpec((Mho, Cout), lambda c: (c, 0)),
            scratch_shapes=scratch,
        ),
        compiler_params=pltpu.CompilerParams(
            dimension_semantics=("parallel",),
            vmem_limit_bytes=_vlim(est),
        ),
    )(h, *big_args, *small_args)
    return out


def _fc_kernel(x_ref, w_ref, s_ref, t_ref, o_ref, *, N, HW):
    feat = x_ref[...].astype(jnp.float32).reshape(N, HW, x_ref.shape[-1])
    feat = feat.mean(axis=1)
    y = jnp.dot(feat.astype(jnp.bfloat16), w_ref[...],
                preferred_element_type=jnp.float32)
    y = y * s_ref[...] + t_ref[...]
    o_ref[...] = jax.nn.sigmoid(y)


def _fc_call(h, w, s, t, N, HW):
    M, C = h.shape
    Cout = w.shape[1]
    est = (_nbytes((M, C), jnp.bfloat16) + _nbytes(w.shape, jnp.bfloat16)
           + _nbytes((N, Cout), jnp.float32))
    return pl.pallas_call(
        functools.partial(_fc_kernel, N=N, HW=HW),
        grid_spec=pltpu.PrefetchScalarGridSpec(
            num_scalar_prefetch=0,
            grid=(1,),
            in_specs=[
                pl.BlockSpec((M, C), lambda i: (0, 0)),
                pl.BlockSpec((C, Cout), lambda i: (0, 0)),
                pl.BlockSpec((1, Cout), lambda i: (0, 0)),
                pl.BlockSpec((1, Cout), lambda i: (0, 0)),
            ],
            out_specs=pl.BlockSpec((N, Cout), lambda i: (0, 0)),
        ),
        out_shape=jax.ShapeDtypeStruct((N, Cout), jnp.float32),
        compiler_params=pltpu.CompilerParams(
            dimension_semantics=("arbitrary",),
            vmem_limit_bytes=_vlim(est),
        ),
    )(h, w, s, t)


def _stem_prep(x, w):
    xh = jnp.transpose(x, (0, 2, 3, 1)).astype(jnp.bfloat16)
    N, H, W, _ = xh.shape
    xp = jnp.pad(xh, ((0, 0), (3, 3), (3, 3), (0, 0)))
    Hp = (H + 6) // 2
    xsd = xp.reshape(N, Hp, 2, Hp, 2, 3).transpose(0, 1, 3, 2, 4, 5)
    xsd = xsd.reshape(N, Hp, Hp, 12)
    perm = []
    for di in range(4):
        for dj in range(4):
            for p in range(2):
                for q in range(2):
                    for c in range(3):
                        i, j = 2 * di + p, 2 * dj + q
                        perm.append(21 * i + 3 * j + c
                                    if (i < 7 and j < 7) else w.shape[0] - 1)
    w16 = w[jnp.array(perm, jnp.int32)].reshape(16, 12, w.shape[1])
    return xsd, w16, H // 2, W // 2


def kernel(*args):
    a = list(args)
    x = a[0]
    conv1 = a[1:4]
    idx = 4
    nblocks = [3, 4, 6, 3]
    layers = []
    for L in range(4):
        blocks = []
        for b in range(nblocks[L]):
            c1 = a[idx:idx + 3]
            c2 = a[idx + 3:idx + 6]
            c3 = a[idx + 6:idx + 9]
            idx += 9
            down = None
            if b == 0:
                down = a[idx:idx + 3]
                idx += 3
            stride = 2 if (L > 0 and b == 0) else 1
            blocks.append((c1, c2, c3, down, stride))
        layers.append(blocks)
    fc_w, fc_scale, fc_shift = a[idx:idx + 3]

    N = x.shape[0]
    xsd, w16, Ho, Wo = _stem_prep(x, conv1[0])
    h = _stem_call(xsd, w16, conv1[1], conv1[2], N, Ho, Wo)
    H = W = Ho // 2
    for blocks in layers:
        h = _stage_call(h, blocks, N, H, W)
        stride0 = blocks[0][4]
        H, W = H // stride0, W // stride0
    out = _fc_call(h, fc_w, fc_scale, fc_shift, N, H * W)
    return out[:, :1]


# speedup vs baseline: 2.1150x; 1.1577x over previous
import functools

import jax
import jax.numpy as jnp
from jax.experimental import pallas as pl
from jax.experimental.pallas import tpu as pltpu


def _nbytes(shape, dtype):
    n = 1
    for d in shape:
        n *= int(d)
    return n * jnp.dtype(dtype).itemsize


def _vlim(est_bytes):
    est = int(1.3 * est_bytes) + (6 << 20)
    return min(max(est, 16 * 1024 * 1024), 56 * 1024 * 1024)


def _stem_kernel(a_ref, w_ref, s_ref, t_ref, o_ref, pp_ref, *, Nh, Ho, Wo):
    y = None
    for di in range(4):
        for dj in range(4):
            tap = a_ref[:, di:di + Ho, dj:dj + Wo, :].reshape(Nh * Ho * Wo, 12)
            d = jnp.dot(tap, w_ref[di * 4 + dj],
                        preferred_element_type=jnp.float32)
            y = d if y is None else y + d
    y = jnp.maximum(y * s_ref[...] + t_ref[...], 0.0).astype(jnp.bfloat16)
    C = y.shape[-1]
    Hp, Wp = Ho // 2, Wo // 2
    pp_ref[...] = jnp.zeros(pp_ref.shape, pp_ref.dtype)
    pp_ref[:, 1:Ho + 1, 1:Wo + 1, :] = y.reshape(Nh, Ho, Wo, C)
    rm = jnp.maximum(jnp.maximum(pp_ref[:, 0:Ho, :, :], pp_ref[:, 1:Ho + 1, :, :]),
                     pp_ref[:, 2:Ho + 2, :, :])
    re = rm.reshape(Nh, Hp, 2, Wo + 2, C)[:, :, 0]
    cm = jnp.maximum(jnp.maximum(re[:, :, 0:Wo, :], re[:, :, 1:Wo + 1, :]),
                     re[:, :, 2:Wo + 2, :])
    ce = cm.reshape(Nh, Hp, Wp, 2, C)[:, :, :, 0]
    o_ref[...] = ce.reshape(Nh * Hp * Wp, C)


def _stem_call(a, w16, s, t, N, Ho, Wo):
    C = w16.shape[2]
    Nh = N // 2
    Hp, Wp = Ho // 2, Wo // 2
    Mo = N * Hp * Wp
    est = (_nbytes(a.shape, jnp.bfloat16) + _nbytes(w16.shape, jnp.bfloat16)
           + _nbytes((Nh, Ho + 2, Wo + 2, C), jnp.bfloat16)
           + _nbytes((Mo // 2, C), jnp.bfloat16) * 2)
    return pl.pallas_call(
        functools.partial(_stem_kernel, Nh=Nh, Ho=Ho, Wo=Wo),
        out_shape=jax.ShapeDtypeStruct((Mo, C), jnp.bfloat16),
        grid_spec=pltpu.PrefetchScalarGridSpec(
            num_scalar_prefetch=0,
            grid=(2,),
            in_specs=[
                pl.BlockSpec((Nh, Ho + 3, Wo + 3, 12), lambda c: (c, 0, 0, 0)),
                pl.BlockSpec(w16.shape, lambda c: (0, 0, 0)),
                pl.BlockSpec((1, C), lambda c: (0, 0)),
                pl.BlockSpec((1, C), lambda c: (0, 0)),
            ],
            out_specs=pl.BlockSpec((Mo // 2, C), lambda c: (c, 0)),
            scratch_shapes=[pltpu.VMEM((Nh, Ho + 2, Wo + 2, C), jnp.bfloat16)],
        ),
        compiler_params=pltpu.CompilerParams(
            dimension_semantics=("parallel",),
            vmem_limit_bytes=_vlim(est),
        ),
    )(a, w16, s, t)


def _block_compute(xv, wr, pad_ref, col_ref, *, Nh, H, W, P, Cin, stride,
                   has_down):
    if has_down:
        w1, s1, t1, w2, s2, t2, w3, s3, t3, wd, sd, td = wr
    else:
        w1, s1, t1, w2, s2, t2, w3, s3, t3 = wr
    Ho, Wo = H // stride, W // stride

    y1 = jnp.dot(xv, w1[...], preferred_element_type=jnp.float32)
    y1 = jnp.maximum(y1 * s1 + t1, 0.0).astype(jnp.bfloat16)

    pad_ref[:, 1:H + 1, 1:W + 1, :] = y1.reshape(Nh, H, W, P)
    taps = []
    for di in range(3):
        for dj in range(3):
            tap = pad_ref[:, di:di + H, dj:dj + W, :]
            if stride == 2:
                tap = tap.reshape(Nh, Ho, 2, Wo, 2, P)[:, :, 0, :, 0, :]
            taps.append(tap.reshape(Nh * Ho * Wo, P))
    if col_ref is not None:
        for ti, tp in enumerate(taps):
            col_ref[:, ti * P:(ti + 1) * P] = tp
        y2 = jnp.dot(col_ref[...], w2[0:9 * P, :],
                     preferred_element_type=jnp.float32)
    else:
        y2 = None
        for ti, tp in enumerate(taps):
            d = jnp.dot(tp, w2[ti * P:(ti + 1) * P, :],
                        preferred_element_type=jnp.float32)
            y2 = d if y2 is None else y2 + d
    y2 = jnp.maximum(y2 * s2 + t2, 0.0).astype(jnp.bfloat16)

    y3 = jnp.dot(y2, w3[...], preferred_element_type=jnp.float32)
    y3 = y3 * s3 + t3

    if has_down:
        if stride == 2:
            xd = xv.reshape(Nh, Ho, 2, Wo, 2, Cin)[:, :, 0, :, 0, :]
            xd = xd.reshape(Nh * Ho * Wo, Cin)
        else:
            xd = xv
        r = jnp.dot(xd, wd[...], preferred_element_type=jnp.float32)
        r = r * sd + td
    else:
        r = xv.astype(jnp.float32)
    return jnp.maximum(y3 + r, 0.0).astype(jnp.bfloat16)


def _stage_kernel(*refs, cfgs, Nh, use_col, share_pad):
    nb = len(cfgs)
    nbig = sum(4 if hd else 3 for (_, _, _, _, _, hd) in cfgs)
    x_ref = refs[0]
    pos = 1
    hbm_ws = refs[pos:pos + nbig]
    pos += nbig
    sc_ref = refs[pos]
    pos += 1
    o_ref = refs[pos]
    pos += 1
    vmem_ws = refs[pos:pos + nbig]
    pos += nbig
    sem = refs[pos]
    act_ref = refs[pos + 1]
    pad0_ref = refs[pos + 2]
    padt_ref = pad0_ref if share_pad else refs[pos + 3]
    col_ref = refs[pos + (3 if share_pad else 4)] if use_col else None

    copies = []
    for k in range(nbig):
        cp = pltpu.make_async_copy(hbm_ws[k], vmem_ws[k], sem.at[k])
        cp.start()
        copies.append(cp)

    pad0_ref[...] = jnp.zeros(pad0_ref.shape, pad0_ref.dtype)
    if not share_pad:
        padt_ref[...] = jnp.zeros(padt_ref.shape, padt_ref.dtype)

    wi = 0
    sr = 0
    for k in range(nb):
        H, W, P, Cin, stride, has_down = cfgs[k]
        Cout = o_ref.shape[-1]
        nw = 4 if has_down else 3
        for j in range(nw):
            copies[wi + j].wait()
        s1 = sc_ref[sr + 0:sr + 1, 0:P]
        t1 = sc_ref[sr + 1:sr + 2, 0:P]
        s2 = sc_ref[sr + 2:sr + 3, 0:P]
        t2 = sc_ref[sr + 3:sr + 4, 0:P]
        s3 = sc_ref[sr + 4:sr + 5, 0:Cout]
        t3 = sc_ref[sr + 5:sr + 6, 0:Cout]
        if has_down:
            sd = sc_ref[sr + 6:sr + 7, 0:Cout]
            td = sc_ref[sr + 7:sr + 8, 0:Cout]
            wr = (vmem_ws[wi], s1, t1, vmem_ws[wi + 1], s2, t2,
                  vmem_ws[wi + 2], s3, t3, vmem_ws[wi + 3], sd, td)
            sr += 8
        else:
            wr = (vmem_ws[wi], s1, t1, vmem_ws[wi + 1], s2, t2,
                  vmem_ws[wi + 2], s3, t3)
            sr += 6
        wi += nw
        xv = x_ref[...] if k == 0 else act_ref[...]
        out = _block_compute(xv, wr, pad0_ref if k == 0 else padt_ref,
                             col_ref, Nh=Nh, H=H, W=W, P=P, Cin=Cin,
                             stride=stride, has_down=has_down)
        if k == nb - 1:
            o_ref[...] = out
        else:
            act_ref[...] = out


def _stage_call(h, blocks, N, H, W):
    M, Cin0 = h.shape
    nb = len(blocks)
    Nh = N // 2
    Mh = M // 2
    P = blocks[0][0][0].shape[1]
    Cout = blocks[0][2][0].shape[1]
    stride0 = blocks[0][4]
    Ho, Wo = H // stride0, W // stride0
    Mho = N * Ho * Wo // 2
    use_col = (P % 128 == 0)
    share_pad = (stride0 == 1)

    big_args, sc_rows = [], []
    cfgs = []
    est = _nbytes((Mh, Cin0), jnp.bfloat16) * 2
    Hk, Wk, Ck = H, W, Cin0
    for (c1, c2, c3, down, stride) in blocks:
        has_down = down is not None
        cfgs.append((Hk, Wk, P, Ck, stride, has_down))
        bigs = [c1[0], c2[0], c3[0]] + ([down[0]] if has_down else [])
        sm = [c1[1], c1[2], c2[1], c2[2], c3[1], c3[2]]
        if has_down:
            sm += [down[1], down[2]]
        big_args += bigs
        sc_rows += [jnp.pad(arr, ((0, 0), (0, Cout - arr.shape[1])))
                    for arr in sm]
        for arr in bigs:
            est += 2 * _nbytes(arr.shape, arr.dtype)
        Hk, Wk, Ck = Hk // stride, Wk // stride, Cout

    sc_pack = jnp.concatenate(sc_rows, axis=0)
    est += _nbytes(sc_pack.shape, sc_pack.dtype)

    in_specs = ([pl.BlockSpec((Mh, Cin0), lambda c: (c, 0))]
                + [pl.BlockSpec(memory_space=pl.ANY) for _ in big_args]
                + [pl.BlockSpec(sc_pack.shape, lambda c: (0, 0))])

    scratch = [pltpu.VMEM(arr.shape, arr.dtype) for arr in big_args]
    scratch.append(pltpu.SemaphoreType.DMA((len(big_args),)))
    scratch += [
        pltpu.VMEM((Mho, Cout), jnp.bfloat16),
        pltpu.VMEM((Nh, H + 2, W + 2, P), jnp.bfloat16),
    ]
    est += 2 * _nbytes((Mho, Cout), jnp.bfloat16) * 2
    est += _nbytes((Nh, H + 2, W + 2, P), jnp.bfloat16)
    if not share_pad:
        scratch.append(pltpu.VMEM((Nh, Ho + 2, Wo + 2, P), jnp.bfloat16))
        est += _nbytes((Nh, Ho + 2, Wo + 2, P), jnp.bfloat16)
    if use_col:
        scratch.append(pltpu.VMEM((Mho, 9 * P), jnp.bfloat16))
        est += _nbytes((Mho, 9 * P), jnp.bfloat16)

    out = pl.pallas_call(
        functools.partial(_stage_kernel, cfgs=cfgs, Nh=Nh, use_col=use_col,
                          share_pad=share_pad),
        out_shape=jax.ShapeDtypeStruct((N * Ho * Wo, Cout), jnp.bfloat16),
        grid_spec=pltpu.PrefetchScalarGridSpec(
            num_scalar_prefetch=0,
            grid=(2,),
            in_specs=in_specs,
            out_specs=pl.BlockSpec((Mho, Cout), lambda c: (c, 0)),
            scratch_shapes=scratch,
        ),
        compiler_params=pltpu.CompilerParams(
            dimension_semantics=("parallel",),
            vmem_limit_bytes=_vlim(est),
        ),
    )(h, *big_args, sc_pack)
    return out


def _fc_kernel(x_ref, w_ref, s_ref, t_ref, o_ref, *, N, HW):
    feat = x_ref[...].astype(jnp.float32).reshape(N, HW, x_ref.shape[-1])
    feat = feat.mean(axis=1)
    y = jnp.dot(feat.astype(jnp.bfloat16), w_ref[...],
                preferred_element_type=jnp.float32)
    y = y * s_ref[...] + t_ref[...]
    o_ref[...] = jax.nn.sigmoid(y)


def _fc_call(h, w, s, t, N, HW):
    M, C = h.shape
    Cout = w.shape[1]
    est = (_nbytes((M, C), jnp.bfloat16) + _nbytes(w.shape, jnp.bfloat16)
           + _nbytes((N, Cout), jnp.float32))
    return pl.pallas_call(
        functools.partial(_fc_kernel, N=N, HW=HW),
        grid_spec=pltpu.PrefetchScalarGridSpec(
            num_scalar_prefetch=0,
            grid=(1,),
            in_specs=[
                pl.BlockSpec((M, C), lambda i: (0, 0)),
                pl.BlockSpec((C, Cout), lambda i: (0, 0)),
                pl.BlockSpec((1, Cout), lambda i: (0, 0)),
                pl.BlockSpec((1, Cout), lambda i: (0, 0)),
            ],
            out_specs=pl.BlockSpec((N, Cout), lambda i: (0, 0)),
        ),
        out_shape=jax.ShapeDtypeStruct((N, Cout), jnp.float32),
        compiler_params=pltpu.CompilerParams(
            dimension_semantics=("arbitrary",),
            vmem_limit_bytes=_vlim(est),
        ),
    )(h, w, s, t)


def _stem_prep(x, w):
    xh = jnp.transpose(x, (0, 2, 3, 1)).astype(jnp.bfloat16)
    N, H, W, _ = xh.shape
    xp = jnp.pad(xh, ((0, 0), (3, 3), (3, 3), (0, 0)))
    Hp = (H + 6) // 2
    xsd = xp.reshape(N, Hp, 2, Hp, 2, 3).transpose(0, 1, 3, 2, 4, 5)
    xsd = xsd.reshape(N, Hp, Hp, 12)
    perm = []
    for di in range(4):
        for dj in range(4):
            for p in range(2):
                for q in range(2):
                    for c in range(3):
                        i, j = 2 * di + p, 2 * dj + q
                        perm.append(21 * i + 3 * j + c
                                    if (i < 7 and j < 7) else w.shape[0] - 1)
    w16 = w[jnp.array(perm, jnp.int32)].reshape(16, 12, w.shape[1])
    return xsd, w16, H // 2, W // 2


def kernel(*args):
    a = list(args)
    x = a[0]
    conv1 = a[1:4]
    idx = 4
    nblocks = [3, 4, 6, 3]
    layers = []
    for L in range(4):
        blocks = []
        for b in range(nblocks[L]):
            c1 = a[idx:idx + 3]
            c2 = a[idx + 3:idx + 6]
            c3 = a[idx + 6:idx + 9]
            idx += 9
            down = None
            if b == 0:
                down = a[idx:idx + 3]
                idx += 3
            stride = 2 if (L > 0 and b == 0) else 1
            blocks.append((c1, c2, c3, down, stride))
        layers.append(blocks)
    fc_w, fc_scale, fc_shift = a[idx:idx + 3]

    N = x.shape[0]
    xsd, w16, Ho, Wo = _stem_prep(x, conv1[0])
    h = _stem_call(xsd, w16, conv1[1], conv1[2], N, Ho, Wo)
    H = W = Ho // 2
    for blocks in layers:
        h = _stage_call(h, blocks, N, H, W)
        stride0 = blocks[0][4]
        H, W = H // stride0, W // stride0
    out = _fc_call(h, fc_w, fc_scale, fc_shift, N, H * W)
    return out[:, :1]


# speedup vs baseline: 2.1187x; 1.0017x over previous
import functools

import jax
import jax.numpy as jnp
from jax.experimental import pallas as pl
from jax.experimental.pallas import tpu as pltpu


def _nbytes(shape, dtype):
    n = 1
    for d in shape:
        n *= int(d)
    return n * jnp.dtype(dtype).itemsize


def _vlim(est_bytes):
    est = int(1.3 * est_bytes) + (6 << 20)
    return min(max(est, 16 * 1024 * 1024), 56 * 1024 * 1024)


def _stem_kernel(a_ref, w_ref, s_ref, t_ref, o_ref, pp_ref, *, Nh, Ho, Wo):
    y = None
    for di in range(4):
        for dj in range(4):
            tap = a_ref[:, di:di + Ho, dj:dj + Wo, :].reshape(Nh * Ho * Wo, 12)
            d = jnp.dot(tap, w_ref[di * 4 + dj],
                        preferred_element_type=jnp.float32)
            y = d if y is None else y + d
    y = jnp.maximum(y * s_ref[...] + t_ref[...], 0.0).astype(jnp.bfloat16)
    C = y.shape[-1]
    Hp, Wp = Ho // 2, Wo // 2
    pp_ref[...] = jnp.zeros(pp_ref.shape, pp_ref.dtype)
    pp_ref[:, 1:Ho + 1, 1:Wo + 1, :] = y.reshape(Nh, Ho, Wo, C)
    rm = jnp.maximum(jnp.maximum(pp_ref[:, 0:Ho, :, :], pp_ref[:, 1:Ho + 1, :, :]),
                     pp_ref[:, 2:Ho + 2, :, :])
    re = rm.reshape(Nh, Hp, 2, Wo + 2, C)[:, :, 0]
    cm = jnp.maximum(jnp.maximum(re[:, :, 0:Wo, :], re[:, :, 1:Wo + 1, :]),
                     re[:, :, 2:Wo + 2, :])
    ce = cm.reshape(Nh, Hp, Wp, 2, C)[:, :, :, 0]
    o_ref[...] = ce.reshape(Nh * Hp * Wp, C)


def _stem_call(a, w16, s, t, N, Ho, Wo):
    C = w16.shape[2]
    Nh = N // 2
    Hp, Wp = Ho // 2, Wo // 2
    Mo = N * Hp * Wp
    est = (_nbytes(a.shape, jnp.bfloat16) + _nbytes(w16.shape, jnp.bfloat16)
           + _nbytes((Nh, Ho + 2, Wo + 2, C), jnp.bfloat16)
           + _nbytes((Mo // 2, C), jnp.bfloat16) * 2)
    return pl.pallas_call(
        functools.partial(_stem_kernel, Nh=Nh, Ho=Ho, Wo=Wo),
        out_shape=jax.ShapeDtypeStruct((Mo, C), jnp.bfloat16),
        grid_spec=pltpu.PrefetchScalarGridSpec(
            num_scalar_prefetch=0,
            grid=(2,),
            in_specs=[
                pl.BlockSpec((Nh, Ho + 3, Wo + 3, 12), lambda c: (c, 0, 0, 0)),
                pl.BlockSpec(w16.shape, lambda c: (0, 0, 0)),
                pl.BlockSpec((1, C), lambda c: (0, 0)),
                pl.BlockSpec((1, C), lambda c: (0, 0)),
            ],
            out_specs=pl.BlockSpec((Mo // 2, C), lambda c: (c, 0)),
            scratch_shapes=[pltpu.VMEM((Nh, Ho + 2, Wo + 2, C), jnp.bfloat16)],
        ),
        compiler_params=pltpu.CompilerParams(
            dimension_semantics=("parallel",),
            vmem_limit_bytes=_vlim(est),
        ),
    )(a, w16, s, t)


def _block_compute(xv, wr, pad_ref, col_ref, *, Nh, H, W, P, Cin, stride,
                   has_down):
    if has_down:
        w1, s1, t1, w2, s2, t2, w3, s3, t3, wd, sd, td = wr
    else:
        w1, s1, t1, w2, s2, t2, w3, s3, t3 = wr
    Ho, Wo = H // stride, W // stride

    y1 = jnp.dot(xv, w1[...], preferred_element_type=jnp.float32)
    y1 = jnp.maximum(y1 * s1 + t1, 0.0).astype(jnp.bfloat16)

    pad_ref[:, 1:H + 1, 1:W + 1, :] = y1.reshape(Nh, H, W, P)
    taps = []
    for di in range(3):
        for dj in range(3):
            tap = pad_ref[:, di:di + H, dj:dj + W, :]
            if stride == 2:
                tap = tap.reshape(Nh, Ho, 2, Wo, 2, P)[:, :, 0, :, 0, :]
            taps.append(tap.reshape(Nh * Ho * Wo, P))
    if col_ref is not None:
        for ti, tp in enumerate(taps):
            col_ref[:, ti * P:(ti + 1) * P] = tp
        y2 = jnp.dot(col_ref[...], w2[0:9 * P, :],
                     preferred_element_type=jnp.float32)
    else:
        y2 = None
        for ti, tp in enumerate(taps):
            d = jnp.dot(tp, w2[ti * P:(ti + 1) * P, :],
                        preferred_element_type=jnp.float32)
            y2 = d if y2 is None else y2 + d
    y2 = jnp.maximum(y2 * s2 + t2, 0.0).astype(jnp.bfloat16)

    y3 = jnp.dot(y2, w3[...], preferred_element_type=jnp.float32)
    y3 = y3 * s3 + t3

    if has_down:
        if stride == 2:
            xd = xv.reshape(Nh, Ho, 2, Wo, 2, Cin)[:, :, 0, :, 0, :]
            xd = xd.reshape(Nh * Ho * Wo, Cin)
        else:
            xd = xv
        r = jnp.dot(xd, wd[...], preferred_element_type=jnp.float32)
        r = r * sd + td
    else:
        r = xv.astype(jnp.float32)
    return jnp.maximum(y3 + r, 0.0).astype(jnp.bfloat16)


def _stage_kernel(*refs, cfgs, Nh, use_col, share_pad):
    nb = len(cfgs)
    nbig = sum(4 if hd else 3 for (_, _, _, _, _, hd) in cfgs)
    x_ref = refs[0]
    pos = 1
    hbm_ws = refs[pos:pos + nbig]
    pos += nbig
    sc_ref = refs[pos]
    pos += 1
    o_ref = refs[pos]
    pos += 1
    vmem_ws = refs[pos:pos + nbig]
    pos += nbig
    sem = refs[pos]
    act_ref = refs[pos + 1]
    pad0_ref = refs[pos + 2]
    padt_ref = pad0_ref if share_pad else refs[pos + 3]
    col_ref = refs[pos + (3 if share_pad else 4)] if use_col else None

    copies = []
    for k in range(nbig):
        cp = pltpu.make_async_copy(hbm_ws[k], vmem_ws[k], sem.at[k])
        cp.start()
        copies.append(cp)

    pad0_ref[...] = jnp.zeros(pad0_ref.shape, pad0_ref.dtype)
    if not share_pad:
        padt_ref[...] = jnp.zeros(padt_ref.shape, padt_ref.dtype)

    wi = 0
    sr = 0
    for k in range(nb):
        H, W, P, Cin, stride, has_down = cfgs[k]
        Cout = o_ref.shape[-1]
        nw = 4 if has_down else 3
        for j in range(nw):
            copies[wi + j].wait()
        s1 = sc_ref[sr + 0:sr + 1, 0:P]
        t1 = sc_ref[sr + 1:sr + 2, 0:P]
        s2 = sc_ref[sr + 2:sr + 3, 0:P]
        t2 = sc_ref[sr + 3:sr + 4, 0:P]
        s3 = sc_ref[sr + 4:sr + 5, 0:Cout]
        t3 = sc_ref[sr + 5:sr + 6, 0:Cout]
        if has_down:
            sd = sc_ref[sr + 6:sr + 7, 0:Cout]
            td = sc_ref[sr + 7:sr + 8, 0:Cout]
            wr = (vmem_ws[wi], s1, t1, vmem_ws[wi + 1], s2, t2,
                  vmem_ws[wi + 2], s3, t3, vmem_ws[wi + 3], sd, td)
            sr += 8
        else:
            wr = (vmem_ws[wi], s1, t1, vmem_ws[wi + 1], s2, t2,
                  vmem_ws[wi + 2], s3, t3)
            sr += 6
        wi += nw
        xv = x_ref[...] if k == 0 else act_ref[...]
        out = _block_compute(xv, wr, pad0_ref if k == 0 else padt_ref,
                             col_ref, Nh=Nh, H=H, W=W, P=P, Cin=Cin,
                             stride=stride, has_down=has_down)
        if k == nb - 1:
            o_ref[...] = out
        else:
            act_ref[...] = out


def _stage_call(h, blocks, N, H, W):
    M, Cin0 = h.shape
    nb = len(blocks)
    Nh = N // 2
    Mh = M // 2
    P = blocks[0][0][0].shape[1]
    Cout = blocks[0][2][0].shape[1]
    stride0 = blocks[0][4]
    Ho, Wo = H // stride0, W // stride0
    Mho = N * Ho * Wo // 2
    use_col = (P % 128 == 0)
    share_pad = (stride0 == 1)

    big_args, sc_rows = [], []
    cfgs = []
    est = _nbytes((Mh, Cin0), jnp.bfloat16) * 2
    Hk, Wk, Ck = H, W, Cin0
    for (c1, c2, c3, down, stride) in blocks:
        has_down = down is not None
        cfgs.append((Hk, Wk, P, Ck, stride, has_down))
        bigs = [c1[0], c2[0], c3[0]] + ([down[0]] if has_down else [])
        sm = [c1[1], c1[2], c2[1], c2[2], c3[1], c3[2]]
        if has_down:
            sm += [down[1], down[2]]
        big_args += bigs
        sc_rows += [jnp.pad(arr, ((0, 0), (0, Cout - arr.shape[1])))
                    for arr in sm]
        for arr in bigs:
            est += 2 * _nbytes(arr.shape, arr.dtype)
        Hk, Wk, Ck = Hk // stride, Wk // stride, Cout

    sc_pack = jnp.concatenate(sc_rows, axis=0)
    est += _nbytes(sc_pack.shape, sc_pack.dtype)

    in_specs = ([pl.BlockSpec((Mh, Cin0), lambda c: (c, 0))]
                + [pl.BlockSpec(memory_space=pl.ANY) for _ in big_args]
                + [pl.BlockSpec(sc_pack.shape, lambda c: (0, 0))])

    scratch = [pltpu.VMEM(arr.shape, arr.dtype) for arr in big_args]
    scratch.append(pltpu.SemaphoreType.DMA((len(big_args),)))
    scratch += [
        pltpu.VMEM((Mho, Cout), jnp.bfloat16),
        pltpu.VMEM((Nh, H + 2, W + 2, P), jnp.bfloat16),
    ]
    est += 2 * _nbytes((Mho, Cout), jnp.bfloat16) * 2
    est += _nbytes((Nh, H + 2, W + 2, P), jnp.bfloat16)
    if not share_pad:
        scratch.append(pltpu.VMEM((Nh, Ho + 2, Wo + 2, P), jnp.bfloat16))
        est += _nbytes((Nh, Ho + 2, Wo + 2, P), jnp.bfloat16)
    if use_col:
        scratch.append(pltpu.VMEM((Mho, 9 * P), jnp.bfloat16))
        est += _nbytes((Mho, 9 * P), jnp.bfloat16)

    out = pl.pallas_call(
        functools.partial(_stage_kernel, cfgs=cfgs, Nh=Nh, use_col=use_col,
                          share_pad=share_pad),
        out_shape=jax.ShapeDtypeStruct((N * Ho * Wo, Cout), jnp.bfloat16),
        grid_spec=pltpu.PrefetchScalarGridSpec(
            num_scalar_prefetch=0,
            grid=(2,),
            in_specs=in_specs,
            out_specs=pl.BlockSpec((Mho, Cout), lambda c: (c, 0)),
            scratch_shapes=scratch,
        ),
        compiler_params=pltpu.CompilerParams(
            dimension_semantics=("parallel",),
            vmem_limit_bytes=_vlim(est),
        ),
    )(h, *big_args, sc_pack)
    return out


def _fc_kernel(x_ref, w_ref, s_ref, t_ref, o_ref, *, N, HW):
    feat = x_ref[...].astype(jnp.float32).reshape(N, HW, x_ref.shape[-1])
    feat = feat.mean(axis=1)
    y = jnp.dot(feat.astype(jnp.bfloat16), w_ref[...],
                preferred_element_type=jnp.float32)
    y = y * s_ref[...] + t_ref[...]
    o_ref[...] = jax.nn.sigmoid(y)


def _fc_call(h, w, s, t, N, HW):
    M, C = h.shape
    Cout = w.shape[1]
    est = (_nbytes((M, C), jnp.bfloat16) + _nbytes(w.shape, jnp.bfloat16)
           + _nbytes((N, Cout), jnp.float32))
    return pl.pallas_call(
        functools.partial(_fc_kernel, N=N, HW=HW),
        grid_spec=pltpu.PrefetchScalarGridSpec(
            num_scalar_prefetch=0,
            grid=(1,),
            in_specs=[
                pl.BlockSpec((M, C), lambda i: (0, 0)),
                pl.BlockSpec((C, Cout), lambda i: (0, 0)),
                pl.BlockSpec((1, Cout), lambda i: (0, 0)),
                pl.BlockSpec((1, Cout), lambda i: (0, 0)),
            ],
            out_specs=pl.BlockSpec((N, Cout), lambda i: (0, 0)),
        ),
        out_shape=jax.ShapeDtypeStruct((N, Cout), jnp.float32),
        compiler_params=pltpu.CompilerParams(
            dimension_semantics=("arbitrary",),
            vmem_limit_bytes=_vlim(est),
        ),
    )(h, w, s, t)


def _stem_prep(x, w):
    N, _, H, W = x.shape
    xp = jnp.pad(x.astype(jnp.bfloat16), ((0, 0), (0, 0), (3, 3), (3, 3)))
    Hp = (H + 6) // 2
    xsd = xp.reshape(N, 3, Hp, 2, Hp, 2).transpose(0, 2, 4, 3, 5, 1)
    xsd = xsd.reshape(N, Hp, Hp, 12)
    perm = []
    for di in range(4):
        for dj in range(4):
            for p in range(2):
                for q in range(2):
                    for c in range(3):
                        i, j = 2 * di + p, 2 * dj + q
                        perm.append(21 * i + 3 * j + c
                                    if (i < 7 and j < 7) else w.shape[0] - 1)
    w16 = w[jnp.array(perm, jnp.int32)].reshape(16, 12, w.shape[1])
    return xsd, w16, H // 2, W // 2


def kernel(*args):
    a = list(args)
    x = a[0]
    conv1 = a[1:4]
    idx = 4
    nblocks = [3, 4, 6, 3]
    layers = []
    for L in range(4):
        blocks = []
        for b in range(nblocks[L]):
            c1 = a[idx:idx + 3]
            c2 = a[idx + 3:idx + 6]
            c3 = a[idx + 6:idx + 9]
            idx += 9
            down = None
            if b == 0:
                down = a[idx:idx + 3]
                idx += 3
            stride = 2 if (L > 0 and b == 0) else 1
            blocks.append((c1, c2, c3, down, stride))
        layers.append(blocks)
    fc_w, fc_scale, fc_shift = a[idx:idx + 3]

    N = x.shape[0]
    xsd, w16, Ho, Wo = _stem_prep(x, conv1[0])
    h = _stem_call(xsd, w16, conv1[1], conv1[2], N, Ho, Wo)
    H = W = Ho // 2
    for blocks in layers:
        h = _stage_call(h, blocks, N, H, W)
        stride0 = blocks[0][4]
        H, W = H // stride0, W // stride0
    out = _fc_call(h, fc_w, fc_scale, fc_shift, N, H * W)
    return out[:, :1]


# speedup vs baseline: 2.2640x; 1.0686x over previous
import functools

import jax
import jax.numpy as jnp
from jax.experimental import pallas as pl
from jax.experimental.pallas import tpu as pltpu


def _nbytes(shape, dtype):
    n = 1
    for d in shape:
        n *= int(d)
    return n * jnp.dtype(dtype).itemsize


def _vlim(est_bytes):
    est = int(1.3 * est_bytes) + (6 << 20)
    return min(max(est, 16 * 1024 * 1024), 56 * 1024 * 1024)


def _stem_kernel(a_ref, w_ref, s_ref, t_ref, o_ref, pp_ref, *, Nh, Ho, Wo):
    y = None
    for di in range(4):
        for dj in range(4):
            tap = a_ref[:, di:di + Ho, dj:dj + Wo, :].reshape(Nh * Ho * Wo, 12)
            d = jnp.dot(tap, w_ref[di * 4 + dj],
                        preferred_element_type=jnp.float32)
            y = d if y is None else y + d
    y = jnp.maximum(y * s_ref[...] + t_ref[...], 0.0).astype(jnp.bfloat16)
    C = y.shape[-1]
    Hp, Wp = Ho // 2, Wo // 2
    pp_ref[...] = jnp.zeros(pp_ref.shape, pp_ref.dtype)
    pp_ref[:, 1:Ho + 1, 1:Wo + 1, :] = y.reshape(Nh, Ho, Wo, C)
    rm = jnp.maximum(jnp.maximum(pp_ref[:, 0:Ho, :, :], pp_ref[:, 1:Ho + 1, :, :]),
                     pp_ref[:, 2:Ho + 2, :, :])
    re = rm.reshape(Nh, Hp, 2, Wo + 2, C)[:, :, 0]
    cm = jnp.maximum(jnp.maximum(re[:, :, 0:Wo, :], re[:, :, 1:Wo + 1, :]),
                     re[:, :, 2:Wo + 2, :])
    ce = cm.reshape(Nh, Hp, Wp, 2, C)[:, :, :, 0]
    o_ref[...] = ce.reshape(Nh * Hp * Wp, C)


def _stem_call(a, w16, s, t, N, Ho, Wo):
    C = w16.shape[2]
    Nh = N // 2
    Hp, Wp = Ho // 2, Wo // 2
    Mo = N * Hp * Wp
    est = (_nbytes(a.shape, jnp.bfloat16) + _nbytes(w16.shape, jnp.bfloat16)
           + _nbytes((Nh, Ho + 2, Wo + 2, C), jnp.bfloat16)
           + _nbytes((Mo // 2, C), jnp.bfloat16) * 2)
    return pl.pallas_call(
        functools.partial(_stem_kernel, Nh=Nh, Ho=Ho, Wo=Wo),
        out_shape=jax.ShapeDtypeStruct((Mo, C), jnp.bfloat16),
        grid_spec=pltpu.PrefetchScalarGridSpec(
            num_scalar_prefetch=0,
            grid=(2,),
            in_specs=[
                pl.BlockSpec((Nh, Ho + 3, Wo + 3, 12), lambda c: (c, 0, 0, 0)),
                pl.BlockSpec(w16.shape, lambda c: (0, 0, 0)),
                pl.BlockSpec((1, C), lambda c: (0, 0)),
                pl.BlockSpec((1, C), lambda c: (0, 0)),
            ],
            out_specs=pl.BlockSpec((Mo // 2, C), lambda c: (c, 0)),
            scratch_shapes=[pltpu.VMEM((Nh, Ho + 2, Wo + 2, C), jnp.bfloat16)],
        ),
        compiler_params=pltpu.CompilerParams(
            dimension_semantics=("parallel",),
            vmem_limit_bytes=_vlim(est),
        ),
    )(a, w16, s, t)


def _block_compute(xv, wr, pad_ref, col_ref, *, Nh, H, W, P, Cin, stride,
                   has_down):
    if has_down:
        w1, s1, t1, w2, s2, t2, w3, s3, t3, wd, sd, td = wr
    else:
        w1, s1, t1, w2, s2, t2, w3, s3, t3 = wr
    Ho, Wo = H // stride, W // stride

    y1 = jnp.dot(xv, w1, preferred_element_type=jnp.float32)
    y1 = jnp.maximum(y1 * s1 + t1, 0.0).astype(jnp.bfloat16)

    pad_ref[:, 1:H + 1, 1:W + 1, :] = y1.reshape(Nh, H, W, P)
    taps = []
    for di in range(3):
        for dj in range(3):
            tap = pad_ref[:, di:di + H, dj:dj + W, :]
            if stride == 2:
                tap = tap.reshape(Nh, Ho, 2, Wo, 2, P)[:, :, 0, :, 0, :]
            taps.append(tap.reshape(Nh * Ho * Wo, P))
    if col_ref is not None:
        for ti, tp in enumerate(taps):
            col_ref[:, ti * P:(ti + 1) * P] = tp
        y2 = jnp.dot(col_ref[...], w2[0:9 * P, :],
                     preferred_element_type=jnp.float32)
    else:
        y2 = None
        for ti, tp in enumerate(taps):
            d = jnp.dot(tp, w2[ti * P:(ti + 1) * P, :],
                        preferred_element_type=jnp.float32)
            y2 = d if y2 is None else y2 + d
    y2 = jnp.maximum(y2 * s2 + t2, 0.0).astype(jnp.bfloat16)

    y3 = jnp.dot(y2, w3, preferred_element_type=jnp.float32)
    y3 = y3 * s3 + t3

    if has_down:
        if stride == 2:
            xd = xv.reshape(Nh, Ho, 2, Wo, 2, Cin)[:, :, 0, :, 0, :]
            xd = xd.reshape(Nh * Ho * Wo, Cin)
        else:
            xd = xv
        r = jnp.dot(xd, wd, preferred_element_type=jnp.float32)
        r = r * sd + td
    else:
        r = xv.astype(jnp.float32)
    return jnp.maximum(y3 + r, 0.0).astype(jnp.bfloat16)


def _stage_kernel(*refs, cfgs, Nh, use_col, share_pad, offs):
    nb = len(cfgs)
    nbig = sum(4 if hd else 3 for (_, _, _, _, _, hd) in cfgs)
    nhbm = 1 if offs is not None else nbig
    x_ref = refs[0]
    pos = 1
    hbm_ws = refs[pos:pos + nhbm]
    pos += nhbm
    sc_ref = refs[pos]
    pos += 1
    o_ref = refs[pos]
    pos += 1
    vmem_ws = refs[pos:pos + nhbm]
    pos += nhbm
    sem = refs[pos]
    act_ref = refs[pos + 1]
    pad0_ref = refs[pos + 2]
    padt_ref = pad0_ref if share_pad else refs[pos + 3]
    col_ref = refs[pos + (3 if share_pad else 4)] if use_col else None

    copies = []
    for k in range(len(hbm_ws)):
        cp = pltpu.make_async_copy(hbm_ws[k], vmem_ws[k], sem.at[k])
        cp.start()
        copies.append(cp)

    pad0_ref[...] = jnp.zeros(pad0_ref.shape, pad0_ref.dtype)
    if not share_pad:
        padt_ref[...] = jnp.zeros(padt_ref.shape, padt_ref.dtype)

    wi = 0
    sr = 0
    for k in range(nb):
        H, W, P, Cin, stride, has_down = cfgs[k]
        Cout = o_ref.shape[-1]
        nw = 4 if has_down else 3
        if offs is not None:
            if k == 0:
                copies[0].wait()
            wv = [vmem_ws[0][ro:ro + rr, 0:cc]
                  for (ro, rr, cc) in offs[wi:wi + nw]]
        else:
            for j in range(nw):
                copies[wi + j].wait()
            wv = [vmem_ws[wi + j][...] for j in range(nw)]
        s1 = sc_ref[sr + 0:sr + 1, 0:P]
        t1 = sc_ref[sr + 1:sr + 2, 0:P]
        s2 = sc_ref[sr + 2:sr + 3, 0:P]
        t2 = sc_ref[sr + 3:sr + 4, 0:P]
        s3 = sc_ref[sr + 4:sr + 5, 0:Cout]
        t3 = sc_ref[sr + 5:sr + 6, 0:Cout]
        if has_down:
            sd = sc_ref[sr + 6:sr + 7, 0:Cout]
            td = sc_ref[sr + 7:sr + 8, 0:Cout]
            wr = (wv[0], s1, t1, wv[1], s2, t2, wv[2], s3, t3,
                  wv[3], sd, td)
            sr += 8
        else:
            wr = (wv[0], s1, t1, wv[1], s2, t2, wv[2], s3, t3)
            sr += 6
        wi += nw
        xv = x_ref[...] if k == 0 else act_ref[...]
        out = _block_compute(xv, wr, pad0_ref if k == 0 else padt_ref,
                             col_ref, Nh=Nh, H=H, W=W, P=P, Cin=Cin,
                             stride=stride, has_down=has_down)
        if k == nb - 1:
            o_ref[...] = out
        else:
            act_ref[...] = out


def _stage_call(h, blocks, N, H, W):
    M, Cin0 = h.shape
    nb = len(blocks)
    Nh = N // 2
    Mh = M // 2
    P = blocks[0][0][0].shape[1]
    Cout = blocks[0][2][0].shape[1]
    stride0 = blocks[0][4]
    Ho, Wo = H // stride0, W // stride0
    Mho = N * Ho * Wo // 2
    use_col = (P % 128 == 0)
    share_pad = (stride0 == 1)

    big_args, sc_rows = [], []
    cfgs = []
    est = _nbytes((Mh, Cin0), jnp.bfloat16) * 2
    Hk, Wk, Ck = H, W, Cin0
    for (c1, c2, c3, down, stride) in blocks:
        has_down = down is not None
        cfgs.append((Hk, Wk, P, Ck, stride, has_down))
        bigs = [c1[0], c2[0], c3[0]] + ([down[0]] if has_down else [])
        sm = [c1[1], c1[2], c2[1], c2[2], c3[1], c3[2]]
        if has_down:
            sm += [down[1], down[2]]
        big_args += bigs
        sc_rows += [jnp.pad(arr, ((0, 0), (0, Cout - arr.shape[1])))
                    for arr in sm]
        for arr in bigs:
            est += 2 * _nbytes(arr.shape, arr.dtype)
        Hk, Wk, Ck = Hk // stride, Wk // stride, Cout

    sc_pack = jnp.concatenate(sc_rows, axis=0)
    est += _nbytes(sc_pack.shape, sc_pack.dtype)

    wmax = max(arr.shape[1] for arr in big_args)
    offs = None
    if wmax <= 512:
        offs, ro = [], 0
        for arr in big_args:
            offs.append((ro, arr.shape[0], arr.shape[1]))
            ro += arr.shape[0]
        wpack = jnp.concatenate(
            [jnp.pad(arr, ((0, 0), (0, wmax - arr.shape[1])))
             for arr in big_args], axis=0)
        big_args = [wpack]

    in_specs = ([pl.BlockSpec((Mh, Cin0), lambda c: (c, 0))]
                + [pl.BlockSpec(memory_space=pl.ANY) for _ in big_args]
                + [pl.BlockSpec(sc_pack.shape, lambda c: (0, 0))])

    scratch = [pltpu.VMEM(arr.shape, arr.dtype) for arr in big_args]
    scratch.append(pltpu.SemaphoreType.DMA((len(big_args),)))
    scratch += [
        pltpu.VMEM((Mho, Cout), jnp.bfloat16),
        pltpu.VMEM((Nh, H + 2, W + 2, P), jnp.bfloat16),
    ]
    est += 2 * _nbytes((Mho, Cout), jnp.bfloat16) * 2
    est += _nbytes((Nh, H + 2, W + 2, P), jnp.bfloat16)
    if not share_pad:
        scratch.append(pltpu.VMEM((Nh, Ho + 2, Wo + 2, P), jnp.bfloat16))
        est += _nbytes((Nh, Ho + 2, Wo + 2, P), jnp.bfloat16)
    if use_col:
        scratch.append(pltpu.VMEM((Mho, 9 * P), jnp.bfloat16))
        est += _nbytes((Mho, 9 * P), jnp.bfloat16)

    out = pl.pallas_call(
        functools.partial(_stage_kernel, cfgs=cfgs, Nh=Nh, use_col=use_col,
                          share_pad=share_pad, offs=offs),
        out_shape=jax.ShapeDtypeStruct((N * Ho * Wo, Cout), jnp.bfloat16),
        grid_spec=pltpu.PrefetchScalarGridSpec(
            num_scalar_prefetch=0,
            grid=(2,),
            in_specs=in_specs,
            out_specs=pl.BlockSpec((Mho, Cout), lambda c: (c, 0)),
            scratch_shapes=scratch,
        ),
        compiler_params=pltpu.CompilerParams(
            dimension_semantics=("parallel",),
            vmem_limit_bytes=_vlim(est),
        ),
    )(h, *big_args, sc_pack)
    return out


def _fc_kernel(x_ref, w_ref, s_ref, t_ref, o_ref, *, N, HW):
    feat = x_ref[...].astype(jnp.float32).reshape(N, HW, x_ref.shape[-1])
    feat = feat.mean(axis=1)
    y = jnp.dot(feat.astype(jnp.bfloat16), w_ref[...],
                preferred_element_type=jnp.float32)
    y = y * s_ref[...] + t_ref[...]
    o_ref[...] = jax.nn.sigmoid(y)


def _fc_call(h, w, s, t, N, HW):
    M, C = h.shape
    Cout = w.shape[1]
    est = (_nbytes((M, C), jnp.bfloat16) + _nbytes(w.shape, jnp.bfloat16)
           + _nbytes((N, Cout), jnp.float32))
    return pl.pallas_call(
        functools.partial(_fc_kernel, N=N, HW=HW),
        grid_spec=pltpu.PrefetchScalarGridSpec(
            num_scalar_prefetch=0,
            grid=(1,),
            in_specs=[
                pl.BlockSpec((M, C), lambda i: (0, 0)),
                pl.BlockSpec((C, Cout), lambda i: (0, 0)),
                pl.BlockSpec((1, Cout), lambda i: (0, 0)),
                pl.BlockSpec((1, Cout), lambda i: (0, 0)),
            ],
            out_specs=pl.BlockSpec((N, Cout), lambda i: (0, 0)),
        ),
        out_shape=jax.ShapeDtypeStruct((N, Cout), jnp.float32),
        compiler_params=pltpu.CompilerParams(
            dimension_semantics=("arbitrary",),
            vmem_limit_bytes=_vlim(est),
        ),
    )(h, w, s, t)


def _stem_prep(x, w):
    N, _, H, W = x.shape
    xp = jnp.pad(x.astype(jnp.bfloat16), ((0, 0), (0, 0), (3, 3), (3, 3)))
    Hp = (H + 6) // 2
    xsd = xp.reshape(N, 3, Hp, 2, Hp, 2).transpose(0, 2, 4, 3, 5, 1)
    xsd = xsd.reshape(N, Hp, Hp, 12)
    perm = []
    for di in range(4):
        for dj in range(4):
            for p in range(2):
                for q in range(2):
                    for c in range(3):
                        i, j = 2 * di + p, 2 * dj + q
                        perm.append(21 * i + 3 * j + c
                                    if (i < 7 and j < 7) else w.shape[0] - 1)
    w16 = w[jnp.array(perm, jnp.int32)].reshape(16, 12, w.shape[1])
    return xsd, w16, H // 2, W // 2


def kernel(*args):
    a = list(args)
    x = a[0]
    conv1 = a[1:4]
    idx = 4
    nblocks = [3, 4, 6, 3]
    layers = []
    for L in range(4):
        blocks = []
        for b in range(nblocks[L]):
            c1 = a[idx:idx + 3]
            c2 = a[idx + 3:idx + 6]
            c3 = a[idx + 6:idx + 9]
            idx += 9
            down = None
            if b == 0:
                down = a[idx:idx + 3]
                idx += 3
            stride = 2 if (L > 0 and b == 0) else 1
            blocks.append((c1, c2, c3, down, stride))
        layers.append(blocks)
    fc_w, fc_scale, fc_shift = a[idx:idx + 3]

    N = x.shape[0]
    xsd, w16, Ho, Wo = _stem_prep(x, conv1[0])
    h = _stem_call(xsd, w16, conv1[1], conv1[2], N, Ho, Wo)
    H = W = Ho // 2
    for blocks in layers:
        h = _stage_call(h, blocks, N, H, W)
        stride0 = blocks[0][4]
        H, W = H // stride0, W // stride0
    out = _fc_call(h, fc_w, fc_scale, fc_shift, N, H * W)
    return out[:, :1]
